# baseline (speedup 1.0000x reference)
import functools

import jax
import jax.numpy as jnp
from jax import lax
from jax.experimental import pallas as pl
from jax.experimental.pallas import tpu as pltpu
from jax.experimental.pallas import tpu_sc as plsc

N_USERS = 1000000
N_ITEMS = 1000000
EMB_DIM = 32
N_FACTORS = 4
DECAY = 1e-3
COR_WEIGHT = 0.01
BATCH_SIZE = 16384
COR_BATCH = 1024

NC, NS = 2, 16
NW = NC * NS
CHUNK = 128
BPW = BATCH_SIZE // NW
CPW = 2 * COR_BATCH // NW

_F32 = jnp.float32


def _sc_gather_body(users_h, pos_h, neg_h, coru_h, cori_h,
                    uall_h, iall_h, uego_h, iego_h,
                    ue_o, pe_o, ne_o, uo_o, po_o, no_o, cu_o, ci_o,
                    idx_v, rows_v, idxs_v, rowss_v, sem):
    wid = lax.axis_index("s") * NC + lax.axis_index("c")

    def run(idx_h, table_outs, bpw, iv, rv, ch):
        base = wid * bpw
        for c in range(bpw // ch):
            off = base + c * ch
            pltpu.sync_copy(idx_h.at[pl.ds(off, ch)], iv)
            for tab_h, out_h in table_outs:
                pltpu.async_copy(tab_h.at[iv], rv, sem).wait()
                pltpu.sync_copy(rv, out_h.at[pl.ds(off, ch)])

    run(users_h, [(uall_h, ue_o), (uego_h, uo_o)], BPW, idx_v, rows_v, CHUNK)
    run(pos_h, [(iall_h, pe_o), (iego_h, po_o)], BPW, idx_v, rows_v, CHUNK)
    run(neg_h, [(iall_h, ne_o), (iego_h, no_o)], BPW, idx_v, rows_v, CHUNK)
    run(coru_h, [(uall_h, cu_o)], COR_BATCH // NW, idxs_v, rowss_v, COR_BATCH // NW)
    run(cori_h, [(iall_h, ci_o)], COR_BATCH // NW, idxs_v, rowss_v, COR_BATCH // NW)


_big = jax.ShapeDtypeStruct((BATCH_SIZE, EMB_DIM), _F32)
_small = jax.ShapeDtypeStruct((COR_BATCH, EMB_DIM), _F32)


@functools.cache
def _sc_gather():
    return pl.kernel(
        _sc_gather_body,
        out_type=(_big, _big, _big, _big, _big, _big, _small, _small),
        mesh=plsc.VectorSubcoreMesh(core_axis_name="c", subcore_axis_name="s",
                                    num_cores=NC, num_subcores=NS),
        scratch_types=[
            pltpu.VMEM((CHUNK,), jnp.int32),
            pltpu.VMEM((CHUNK, EMB_DIM), _F32),
            pltpu.VMEM((COR_BATCH // NW,), jnp.int32),
            pltpu.VMEM((COR_BATCH // NW, EMB_DIM), _F32),
            pltpu.SemaphoreType.DMA,
        ],
        compiler_params=pltpu.CompilerParams(use_tc_tiling_on_sc=False),
    )


N2 = 2 * COR_BATCH
GRID = 8
CB = N2 // GRID
BB = BATCH_SIZE // GRID
FD = EMB_DIM // N_FACTORS

_A_MF = 0
_A_REG = 1
_A_S = 2
_A_SELF = 6
_A_Q = 10
_A_CROSS = 14
_A_P = 17
_N_ACC = 20


def _tc_body(ue, pe, ne, uo, po, no, ui, uiT, out_ref, acc):
    i = pl.program_id(0)

    @pl.when(i == 0)
    def _init():
        for j in range(_N_ACC):
            acc[j] = 0.0

    u = ue[...]
    pos_s = jnp.sum(u * pe[...], axis=1)
    neg_s = jnp.sum(u * ne[...], axis=1)
    d = neg_s - pos_s
    sp = jnp.maximum(d, 0.0) + jnp.log(1.0 + jnp.exp(-jnp.abs(d)))
    acc[_A_MF] = acc[_A_MF] + jnp.sum(sp)
    reg = jnp.sum(uo[...] * uo[...]) + jnp.sum(po[...] * po[...]) \
        + jnp.sum(no[...] * no[...])
    acc[_A_REG] = acc[_A_REG] + reg

    xb = ui[...]
    xt = uiT[...]
    ds = []
    rss = []
    for k in range(N_FACTORS):
        xk = xb[:, k * FD:(k + 1) * FD]
        tk = xt[k * FD:(k + 1) * FD, :]
        r_full = jnp.sum(tk * tk, axis=0)
        r_blk = jnp.sum(xk * xk, axis=1)
        g = jnp.dot(xk, tk, preferred_element_type=_F32)
        d2 = r_blk[:, None] - 2.0 * g + r_full[None, :]
        dmat = jnp.sqrt(jnp.maximum(d2, 0.0) + 1e-8)
        rs = jnp.sum(dmat, axis=1)
        acc[_A_S + k] = acc[_A_S + k] + jnp.sum(rs)
        acc[_A_SELF + k] = acc[_A_SELF + k] + jnp.sum(rs * rs)
        acc[_A_Q + k] = acc[_A_Q + k] + jnp.sum(dmat * dmat)
        ds.append(dmat)
        rss.append(rs)
    for p in range(N_FACTORS - 1):
        acc[_A_CROSS + p] = acc[_A_CROSS + p] + jnp.sum(rss[p] * rss[p + 1])
        acc[_A_P + p] = acc[_A_P + p] + jnp.sum(ds[p] * ds[p + 1])

    @pl.when(i == GRID - 1)
    def _fin():
        n = float(N2)
        mf = acc[_A_MF] / float(BATCH_SIZE)
        emb = DECAY * (acc[_A_REG] / 2.0) / float(BATCH_SIZE)

        def centered_sum(prod, cross, sa, sb):
            return prod - (2.0 / n) * cross + sa * sb / (n * n)

        def dcov(csum):
            return jnp.sqrt(jnp.maximum(csum / (n * n), 0.0) + 1e-8)

        cor = jnp.float32(0.0)
        for p in range(N_FACTORS - 1):
            a, b = p, p + 1
            s12 = centered_sum(acc[_A_P + p], acc[_A_CROSS + p],
                               acc[_A_S + a], acc[_A_S + b])
            s11 = centered_sum(acc[_A_Q + a], acc[_A_SELF + a],
                               acc[_A_S + a], acc[_A_S + a])
            s22 = centered_sum(acc[_A_Q + b], acc[_A_SELF + b],
                               acc[_A_S + b], acc[_A_S + b])
            d12, d11, d22 = dcov(s12), dcov(s11), dcov(s22)
            cor = cor + d12 / (jnp.sqrt(jnp.maximum(d11 * d22, 0.0)) + 1e-10)
        cor_loss = COR_WEIGHT * cor / ((N_FACTORS + 1.0) * N_FACTORS / 2.0)
        out_ref[0] = mf
        out_ref[1] = emb
        out_ref[2] = cor_loss
        out_ref[3] = mf + emb + cor_loss


_big_spec = pl.BlockSpec((BB, EMB_DIM), lambda i: (i, 0))

_tc_losses = pl.pallas_call(
    _tc_body,
    grid=(GRID,),
    in_specs=[
        _big_spec, _big_spec, _big_spec, _big_spec, _big_spec, _big_spec,
        pl.BlockSpec((CB, EMB_DIM), lambda i: (i, 0)),
        pl.BlockSpec((EMB_DIM, N2), lambda i: (0, 0)),
    ],
    out_specs=pl.BlockSpec(memory_space=pltpu.SMEM),
    out_shape=jax.ShapeDtypeStruct((4,), _F32),
    scratch_shapes=[pltpu.SMEM((_N_ACC,), _F32)],
)


def kernel(users, pos_items, neg_items, cor_users, cor_items,
           user_embedding, item_embedding,
           user_all_embeddings, item_all_embeddings):
    users = users.astype(jnp.int32)
    pos_items = pos_items.astype(jnp.int32)
    neg_items = neg_items.astype(jnp.int32)
    cor_users = cor_users.astype(jnp.int32)
    cor_items = cor_items.astype(jnp.int32)

    ue, pe, ne, uo, po, no, cu, ci = _sc_gather()(
        users, pos_items, neg_items, cor_users, cor_items,
        user_all_embeddings, item_all_embeddings,
        user_embedding, item_embedding)

    ui = jnp.concatenate([cu, ci], axis=0)
    out = _tc_losses(ue, pe, ne, uo, po, no, ui, ui.T)
    mf_loss, emb_loss, cor_loss, loss = out[0], out[1], out[2], out[3]
    return (mf_loss, emb_loss, cor_loss, loss)
